# baseline (speedup 1.0000x reference)
import functools

import jax
import jax.numpy as jnp
from jax import lax
from jax.experimental import pallas as pl
from jax.experimental.pallas import tpu as pltpu
from jax.experimental.pallas import tpu_sc as plsc

NC, NS = 2, 16
NW = NC * NS
R = 128


def kernel(x, pe, chrom_table, positions, chromosomes):
    B, S, CX = x.shape
    ML, CPE = pe.shape
    CCH = chrom_table.shape[1]
    CO = CX + CPE + CCH
    SPW = B // NW
    NCHUNK = S // R

    mesh = plsc.VectorSubcoreMesh(core_axis_name="c", subcore_axis_name="s",
                                  num_cores=NC, num_subcores=NS)

    @functools.partial(
        pl.kernel, mesh=mesh,
        out_type=jax.ShapeDtypeStruct((B, S, CO), jnp.float32),
        compiler_params=pltpu.CompilerParams(use_tc_tiling_on_sc=False),
        scratch_types=[
            pltpu.VMEM((B + 16,), jnp.int32),
            pltpu.VMEM((B + 16,), jnp.int32),
            pltpu.VMEM((CCH,), jnp.float32),
            pltpu.VMEM((R, CCH), jnp.float32),
            pltpu.VMEM((R, CX), jnp.float32),
            pltpu.VMEM((R, CPE), jnp.float32),
        ],
    )
    def k(x_hbm, pe_hbm, tbl_hbm, pos_hbm, chr_hbm, out_hbm,
          posv, chrv, rowv, chtile, bufx, bufpe):
        wid = lax.axis_index("s") * NC + lax.axis_index("c")
        pltpu.sync_copy(pos_hbm, posv.at[pl.ds(0, B)])
        pltpu.sync_copy(chr_hbm, chrv.at[pl.ds(0, B)])
        for kk in range(SPW):
            i = wid * SPW + kk
            pos = jnp.clip(posv[pl.ds(i, 16)][0], 0, ML - S)
            c = chrv[pl.ds(i, 16)][0]
            pltpu.sync_copy(tbl_hbm.at[c], rowv)

            @pl.loop(0, R)
            def _fill(rr):
                for g in range(CCH // 16):
                    chtile[rr, pl.ds(g * 16, 16)] = rowv[pl.ds(g * 16, 16)]

            @pl.loop(0, NCHUNK)
            def _chunk(t):
                r0 = t * R
                pltpu.sync_copy(x_hbm.at[pl.ds(i * S + r0, R)], bufx)
                pltpu.sync_copy(bufx, out_hbm.at[i, pl.ds(r0, R), pl.ds(0, CX)])
                pltpu.sync_copy(pe_hbm.at[pl.ds(pos + r0, R)], bufpe)
                pltpu.sync_copy(bufpe,
                                out_hbm.at[i, pl.ds(r0, R), pl.ds(CX, CPE)])
                pltpu.sync_copy(chtile,
                                out_hbm.at[i, pl.ds(r0, R),
                                           pl.ds(CX + CPE, CCH)])

    return k(x.reshape(B * S, CX), pe, chrom_table,
             positions.astype(jnp.int32), chromosomes.astype(jnp.int32))


# speedup vs baseline: 1.8620x; 1.8620x over previous
import functools

import jax
import jax.numpy as jnp
from jax.experimental import pallas as pl
from jax.experimental.pallas import tpu as pltpu

B_BLK = 4


def _pe_concat_kernel(positions_ref, chromosomes_ref,
                      x_ref, pe_ref, chrom_ref, out_ref,
                      *, seq_len, max_len, c_x, c_pe, c_ch):
    i = pl.program_id(0)
    out_ref[:, :, 0:c_x] = x_ref[...]
    tbl = chrom_ref[:, :]
    rows = jax.lax.broadcasted_iota(jnp.int32, tbl.shape, 0)
    for k in range(B_BLK):
        pos = jnp.clip(positions_ref[i * B_BLK + k], 0, max_len - seq_len)
        base = jnp.minimum((pos // 8) * 8, max_len - (seq_len + 8))
        r = pos - base
        slab = pe_ref[pl.ds(base, seq_len + 8), :]
        shift = jnp.where(r == 0, 0, seq_len + 8 - r)
        rolled = pltpu.roll(slab, shift, 0)
        out_ref[k, :, c_x:c_x + c_pe] = rolled[:seq_len, :]
        c = chromosomes_ref[i * B_BLK + k]
        row = jnp.sum(jnp.where(rows == c, tbl, 0.0), axis=0, keepdims=True)
        out_ref[k, :, c_x + c_pe:c_x + c_pe + c_ch] = jnp.broadcast_to(
            row, (seq_len, c_ch))


def kernel(x, pe, chrom_table, positions, chromosomes):
    batch, seq_len, c_x = x.shape
    max_len, c_pe = pe.shape
    c_ch = chrom_table.shape[1]
    c_out = c_x + c_pe + c_ch

    grid_spec = pltpu.PrefetchScalarGridSpec(
        num_scalar_prefetch=2,
        grid=(batch // B_BLK,),
        in_specs=[
            pl.BlockSpec((B_BLK, seq_len, c_x), lambda i, *_: (i, 0, 0)),
            pl.BlockSpec((max_len, c_pe), lambda i, *_: (0, 0)),
            pl.BlockSpec(chrom_table.shape, lambda i, *_: (0, 0)),
        ],
        out_specs=pl.BlockSpec((B_BLK, seq_len, c_out), lambda i, *_: (i, 0, 0)),
    )

    fn = pl.pallas_call(
        functools.partial(_pe_concat_kernel, seq_len=seq_len, max_len=max_len,
                          c_x=c_x, c_pe=c_pe, c_ch=c_ch),
        grid_spec=grid_spec,
        out_shape=jax.ShapeDtypeStruct((batch, seq_len, c_out), x.dtype),
    )
    return fn(positions.astype(jnp.int32), chromosomes.astype(jnp.int32),
              x, pe, chrom_table)
